# Initial kernel scaffold; baseline (speedup 1.0000x reference)
import jax
import jax.numpy as jnp
from jax.experimental import pallas as pl


def kernel(h, edge_index, edge_weight, W0_lin, W0_root, b0, W1_lin, W1_root, b1, Wm, bm, Wr, br):
    raise NotImplementedError("write your pallas kernel here")


# speedup vs baseline: 8.3564x; 8.3564x over previous
import jax
import jax.numpy as jnp
from jax import lax
from jax.experimental import pallas as pl
from jax.experimental.pallas import tpu as pltpu
from jax.experimental.pallas import tpu_sc as plsc

_B = 8
_N = 10000
_E = 320000
_R = _B * _N
_NC = 2
_NS = 16
_K = 80
_HOR = 12
_W0 = 640
_W1 = 400


def _sc_mesh():
    return plsc.VectorSubcoreMesh(
        core_axis_name="c", subcore_axis_name="s",
        num_cores=_NC, num_subcores=_NS)


def _zero_acc(zbuf, acc, s, zrows):
    @pl.when(s < _NS - 1)
    def _():
        for z in range(_W0 // zrows):
            pltpu.sync_copy(zbuf, acc.at[pl.ds(s * _W0 + z * zrows, zrows)])

    @pl.when(s == _NS - 1)
    def _():
        for z in range(_W1 // zrows):
            pltpu.sync_copy(zbuf, acc.at[pl.ds(s * _W0 + z * zrows, zrows)])


def _writeback(acc, out, s, row0):
    @pl.when(s < _NS - 1)
    def _():
        pltpu.sync_copy(acc.at[pl.ds(s * _W0, _W0)],
                        out.at[pl.ds(row0 + s * _W0, _W0)])

    @pl.when(s == _NS - 1)
    def _():
        pltpu.sync_copy(acc.at[pl.ds(s * _W0, _W1)],
                        out.at[pl.ds(row0 + s * _W0, _W1)])


def _make_spmm(feature_split):
    nsl = _NS if feature_split else _NC * _NS
    nch = _E // nsl // _K
    mul = 2 if feature_split else 1
    nb = _B if feature_split else _B + 1
    nrows = _R if feature_split else _R + _N
    zrows = 80

    def body(xi, src_hbm, dst_hbm, ewb_hbm, outlo, outhi,
             srcb, dstc, wbuf, idx_b, rows, acc, gsem, wsem):
        c = lax.axis_index("c")
        s = lax.axis_index("s")
        w = s if feature_split else c * _NS + s
        zv = jnp.zeros((16,), jnp.float32)

        def batch(b, carry):
            def zinit(r, carry0):
                for f in range(8):
                    rows[r, pl.ds(f * 16, 16)] = zv
                return carry0
            lax.fori_loop(0, zrows, zinit, 0)
            _zero_acc(rows, acc, s, zrows)
            plsc.subcore_barrier()
            if feature_split:
                off = b * (2 * _N) + c
            else:
                off = b * _N

            def chunk(i, carry2):
                cps = pltpu.async_copy(src_hbm.at[w, i], srcb, wsem)
                cpd = pltpu.async_copy(dst_hbm.at[w, i], dstc.at[0], wsem)
                cpw = pltpu.async_copy(ewb_hbm.at[w, i], wbuf, wsem)
                cps.wait()
                cpd.wait()
                cpw.wait()
                for t in range(_K // 16):
                    sv = srcb[pl.ds(t * 16, 16)]
                    idx_b[pl.ds(t * 16, 16)] = sv * mul + off
                pltpu.async_copy(xi.at[idx_b], rows, gsem).wait()

                def rowloop(j, carry3):
                    wv = wbuf[j, pl.ds(0, 16)]
                    for f in range(8):
                        sl = pl.ds(f * 16, 16)
                        rows[j, sl] = rows[j, sl] * wv
                    return carry3
                lax.fori_loop(0, _K, rowloop, 0)
                pltpu.sync_copy(rows, acc.at[dstc.at[0]], add=True)
                return carry2
            lax.fori_loop(0, nch, chunk, 0)
            plsc.subcore_barrier()

            @pl.when(c == 0)
            def _():
                _writeback(acc, outlo, s, b * _N)

            @pl.when(c == 1)
            def _():
                _writeback(acc, outhi, s, b * _N)
            plsc.subcore_barrier()
            return carry
        lax.fori_loop(0, _B, batch, 0)

    return pl.kernel(
        body,
        out_type=(jax.ShapeDtypeStruct((_R, 128), jnp.float32),
                  jax.ShapeDtypeStruct((_R, 128), jnp.float32)),
        mesh=_sc_mesh(),
        scratch_types=(
            pltpu.VMEM((_K,), jnp.int32),
            pltpu.VMEM((1, _K), jnp.int32),
            pltpu.VMEM((_K, 16), jnp.float32),
            pltpu.VMEM((_K,), jnp.int32),
            pltpu.VMEM((_K, 128), jnp.float32),
            pltpu.VMEM_SHARED((_N, 128), jnp.float32),
            pltpu.SemaphoreType.DMA,
            pltpu.SemaphoreType.DMA,
        ),
        name="sc_spmm_fs%d" % int(feature_split),
    )


def _make_deg():
    nsl = _NC * _NS
    nch = _E // nsl // _K
    zrows = 80

    def body(dst_hbm, ewb_hbm, outlo, outhi, dstc, wbuf, wfull, acc, wsem):
        c = lax.axis_index("c")
        s = lax.axis_index("s")
        w = c * _NS + s
        zv = jnp.zeros((16,), jnp.float32)

        def zinit(r, carry0):
            for f in range(8):
                wfull[r, pl.ds(f * 16, 16)] = zv
            return carry0
        lax.fori_loop(0, zrows, zinit, 0)
        _zero_acc(wfull, acc, s, zrows)
        plsc.subcore_barrier()

        def chunk(i, carry2):
            cpd = pltpu.async_copy(dst_hbm.at[w, i], dstc.at[0], wsem)
            cpw = pltpu.async_copy(ewb_hbm.at[w, i], wbuf, wsem)
            cpd.wait()
            cpw.wait()

            def rowloop(j, carry3):
                wv = wbuf[j, pl.ds(0, 16)]
                for f in range(8):
                    wfull[j, pl.ds(f * 16, 16)] = wv
                return carry3
            lax.fori_loop(0, _K, rowloop, 0)
            pltpu.sync_copy(wfull, acc.at[dstc.at[0]], add=True)
            return carry2
        lax.fori_loop(0, nch, chunk, 0)
        plsc.subcore_barrier()

        @pl.when(c == 0)
        def _():
            _writeback(acc, outlo, s, 0)

        @pl.when(c == 1)
        def _():
            _writeback(acc, outhi, s, 0)

    return pl.kernel(
        body,
        out_type=(jax.ShapeDtypeStruct((_N, 128), jnp.float32),
                  jax.ShapeDtypeStruct((_N, 128), jnp.float32)),
        mesh=_sc_mesh(),
        scratch_types=(
            pltpu.VMEM((1, _K), jnp.int32),
            pltpu.VMEM((_K, 16), jnp.float32),
            pltpu.VMEM((_K, 128), jnp.float32),
            pltpu.VMEM_SHARED((_N, 128), jnp.float32),
            pltpu.SemaphoreType.DMA,
        ),
        name="sc_deg",
    )


_deg_kernel = _make_deg()
_spmm_l0 = _make_spmm(False)
_spmm_l1 = _make_spmm(True)


_BLK = 2000


def _combine0_body(alo, ahi, hb, sc, wl, wr, bb, out):
    agg = sc[...] * (alo[...] + ahi[...])
    t = jnp.dot(agg, wl[...], preferred_element_type=jnp.float32)
    t = t + jnp.dot(hb[...], wr[...], preferred_element_type=jnp.float32)
    t = jnp.maximum(t + bb[...], 0.0)
    out[...] = t.reshape(2 * _BLK, 128)


def _combine0(a0lo, a0hi, h2, scale, W_lin, W_root, b):
    g = _R // _BLK
    return pl.pallas_call(
        _combine0_body,
        grid=(g,),
        in_specs=[
            pl.BlockSpec((_BLK, 128), lambda i: (i, 0)),
            pl.BlockSpec((_BLK, 128), lambda i: (i, 0)),
            pl.BlockSpec((_BLK, 128), lambda i: (i, 0)),
            pl.BlockSpec((_BLK, 1), lambda i: (i, 0)),
            pl.BlockSpec((128, 256), lambda i: (0, 0)),
            pl.BlockSpec((128, 256), lambda i: (0, 0)),
            pl.BlockSpec((1, 256), lambda i: (0, 0)),
        ],
        out_specs=pl.BlockSpec((2 * _BLK, 128), lambda i: (i, 0)),
        out_shape=jax.ShapeDtypeStruct((2 * _R, 128), jnp.float32),
    )(a0lo, a0hi, h2, scale, W_lin, W_root, b)


def _mlp_body(alo, ahi, x1b, sc, wl, wr, bb, wm, bm_, wrd, brd, out):
    a = jnp.dot(sc[...] * alo[...], wl[0:128, :],
                preferred_element_type=jnp.float32)
    a = a + jnp.dot(sc[...] * ahi[...], wl[128:256, :],
                    preferred_element_type=jnp.float32)
    x1 = x1b[...].reshape(_BLK, 256)
    t = a + jnp.dot(x1, wr[...], preferred_element_type=jnp.float32)
    t = jnp.maximum(t + bb[...], 0.0)
    t2 = jnp.dot(t, wm[...], preferred_element_type=jnp.float32)
    t2 = jnp.maximum(t2 + bm_[...], 0.0)
    o = jnp.dot(t2, wrd[...], preferred_element_type=jnp.float32)
    out[...] = o + brd[...]


def _mlp(a1lo, a1hi, x1i, scale, W_lin, W_root, b1, Wm, bm, Wr, br):
    g = _R // _BLK
    return pl.pallas_call(
        _mlp_body,
        grid=(g,),
        in_specs=[
            pl.BlockSpec((_BLK, 128), lambda i: (i, 0)),
            pl.BlockSpec((_BLK, 128), lambda i: (i, 0)),
            pl.BlockSpec((2 * _BLK, 128), lambda i: (i, 0)),
            pl.BlockSpec((_BLK, 1), lambda i: (i, 0)),
            pl.BlockSpec((256, 256), lambda i: (0, 0)),
            pl.BlockSpec((256, 256), lambda i: (0, 0)),
            pl.BlockSpec((1, 256), lambda i: (0, 0)),
            pl.BlockSpec((256, 256), lambda i: (0, 0)),
            pl.BlockSpec((1, 256), lambda i: (0, 0)),
            pl.BlockSpec((256, _HOR), lambda i: (0, 0)),
            pl.BlockSpec((1, _HOR), lambda i: (0, 0)),
        ],
        out_specs=pl.BlockSpec((_BLK, _HOR), lambda i: (i, 0)),
        out_shape=jax.ShapeDtypeStruct((_R, _HOR), jnp.float32),
    )(a1lo, a1hi, x1i, scale, W_lin, W_root, b1, Wm, bm, Wr, br)


def kernel(h, edge_index, edge_weight, W0_lin, W0_root, b0,
           W1_lin, W1_root, b1, Wm, bm, Wr, br):
    src = edge_index[0]
    dst = edge_index[1]
    ewb = jnp.broadcast_to(edge_weight[:, None], (_E, 16))
    n16 = _E // _NS // _K
    src16 = src.reshape(_NS, n16, _K)
    dst16 = dst.reshape(_NS, n16, _K)
    ewb16 = ewb.reshape(_NS, n16, _K, 16)
    n32 = _E // (_NC * _NS) // _K
    src32 = src.reshape(_NC * _NS, n32, _K)
    dst32 = dst.reshape(_NC * _NS, n32, _K)
    ewb32 = ewb.reshape(_NC * _NS, n32, _K, 16)

    deglo, deghi = _deg_kernel(dst32, ewb32)
    deg = deglo[:, 0] + deghi[:, 0]
    inv = jnp.where(deg > 0, 1.0 / deg, 0.0)
    scale = jnp.broadcast_to(inv[None, :], (_B, _N)).reshape(_R, 1)

    h2 = h.reshape(_R, 128)
    a0lo, a0hi = _spmm_l0(h2, src32, dst32, ewb32)
    x1_il = _combine0(a0lo, a0hi, h2, scale, W0_lin, W0_root,
                      b0.reshape(1, 256))
    a1lo, a1hi = _spmm_l1(x1_il, src16, dst16, ewb16)
    o = _mlp(a1lo, a1hi, x1_il, scale, W1_lin, W1_root, b1.reshape(1, 256),
             Wm, bm.reshape(1, 256), Wr, br.reshape(1, _HOR))
    return o.reshape(_B, _N, _HOR, 1).transpose(0, 2, 1, 3)


# speedup vs baseline: 14.7904x; 1.7700x over previous
import jax
import jax.numpy as jnp
from jax import lax
from jax.experimental import pallas as pl
from jax.experimental.pallas import tpu as pltpu
from jax.experimental.pallas import tpu_sc as plsc

_B = 8
_N = 10000
_E = 320000
_R = _B * _N
_NC = 2
_NS = 16
_K = 80
_HOR = 12
_W0 = 640
_W1 = 400


def _sc_mesh():
    return plsc.VectorSubcoreMesh(
        core_axis_name="c", subcore_axis_name="s",
        num_cores=_NC, num_subcores=_NS)


def _zero_acc(zbuf, acc, s, zrows):
    @pl.when(s < _NS - 1)
    def _():
        for z in range(_W0 // zrows):
            pltpu.sync_copy(zbuf, acc.at[pl.ds(s * _W0 + z * zrows, zrows)])

    @pl.when(s == _NS - 1)
    def _():
        for z in range(_W1 // zrows):
            pltpu.sync_copy(zbuf, acc.at[pl.ds(s * _W0 + z * zrows, zrows)])


def _writeback(acc, out, s, row0):
    @pl.when(s < _NS - 1)
    def _():
        pltpu.sync_copy(acc.at[pl.ds(s * _W0, _W0)],
                        out.at[pl.ds(row0 + s * _W0, _W0)])

    @pl.when(s == _NS - 1)
    def _():
        pltpu.sync_copy(acc.at[pl.ds(s * _W0, _W1)],
                        out.at[pl.ds(row0 + s * _W0, _W1)])


def _make_spmm(feature_split):
    nsl = _NS if feature_split else _NC * _NS
    nch = _E // nsl // _K
    mul = 2 if feature_split else 1
    zrows = 80
    kmax = (nch - 1) // 2
    tail = (nch % 2 == 0)

    def body(xi, src_hbm, dst_hbm, ewb_hbm, outlo, outhi,
             srcb0, srcb1, dstc0, dstc1, sidx0, sidx1, wbuf0, wbuf1,
             idxb0, idxb1, rows0, rows1, acc,
             esem0, esem1, gsem0, gsem1, ssem0, ssem1):
        c = lax.axis_index("c")
        s = lax.axis_index("s")
        w = s if feature_split else c * _NS + s
        zv = jnp.zeros((16,), jnp.float32)
        srcb = (srcb0, srcb1)
        dstc = (dstc0, dstc1)
        sidx = (sidx0, sidx1)
        wbuf = (wbuf0, wbuf1)
        idxb = (idxb0, idxb1)
        rows = (rows0, rows1)
        esem = (esem0, esem1)
        gsem = (gsem0, gsem1)
        ssem = (ssem0, ssem1)

        def e_issue_sd(j, p):
            pltpu.async_copy(src_hbm.at[w, j], srcb[p], esem[p])
            pltpu.async_copy(dst_hbm.at[w, j], dstc[p].at[0], esem[p])

        def e_issue_w(j, p):
            pltpu.async_copy(ewb_hbm.at[w, j], wbuf[p], esem[p])

        def e_wait(p):
            pltpu.make_async_copy(src_hbm.at[w, 0], srcb[p], esem[p]).wait()
            pltpu.make_async_copy(dst_hbm.at[w, 0], dstc[p].at[0],
                                  esem[p]).wait()
            pltpu.make_async_copy(ewb_hbm.at[w, 0], wbuf[p], esem[p]).wait()

        def g_issue(p):
            pltpu.async_copy(xi.at[idxb[p]], rows[p], gsem[p])

        def g_wait(p):
            pltpu.make_async_copy(xi.at[idxb[p]], rows[p], gsem[p]).wait()

        def s_issue(p):
            pltpu.async_copy(rows[p], acc.at[sidx[p].at[0]], ssem[p],
                             add=True)

        def s_wait(p):
            pltpu.make_async_copy(rows[p], acc.at[sidx[p].at[0]],
                                  ssem[p]).wait()

        def scale(p):
            def rowloop(j, carry3):
                wv = wbuf[p][j, pl.ds(0, 16)]
                for f in range(8):
                    sl = pl.ds(f * 16, 16)
                    rows[p][j, sl] = rows[p][j, sl] * wv
                return carry3
            lax.fori_loop(0, _K, rowloop, 0)

        def batch(b, carry):
            def zinit(r, carry0):
                for f in range(8):
                    rows[0][r, pl.ds(f * 16, 16)] = zv
                return carry0
            lax.fori_loop(0, zrows, zinit, 0)
            _zero_acc(rows[0], acc, s, zrows)
            plsc.subcore_barrier()
            if feature_split:
                off = b * (2 * _N) + c
            else:
                off = b * _N

            def build_idx(p):
                for t in range(_K // 16):
                    sv = srcb[p][pl.ds(t * 16, 16)]
                    idxb[p][pl.ds(t * 16, 16)] = sv * mul + off

            def copy_sidx(p):
                for t in range(_K // 16):
                    sidx[p][0, pl.ds(t * 16, 16)] = \
                        dstc[p][0, pl.ds(t * 16, 16)]

            def pref(jn, p, first):
                e_wait(p)
                build_idx(p)
                if not first:
                    s_wait(p)
                copy_sidx(p)
                g_issue(p)

                @pl.when(jn + 2 < nch)
                def _():
                    e_issue_sd(jn + 2, p)

            def proc(j, p, last):
                g_wait(p)
                scale(p)
                if not last:
                    @pl.when(j + 2 < nch)
                    def _():
                        e_issue_w(j + 2, p)
                s_issue(p)

            e_issue_sd(0, 0)
            e_issue_w(0, 0)
            e_issue_sd(1, 1)
            e_issue_w(1, 1)
            pref(0, 0, True)
            pref(1, 1, True)
            proc(0, 0, False)

            def pair(k, carry2):
                j0 = 2 * k - 1

                @pl.when(j0 + 1 < nch)
                def _():
                    pref(j0 + 1, 0, False)
                proc(j0, 1, False)

                @pl.when(j0 + 2 < nch)
                def _():
                    pref(j0 + 2, 1, False)
                proc(j0 + 1, 0, False)
                return carry2
            lax.fori_loop(1, kmax + 1, pair, 0)
            if tail:
                proc(nch - 1, 1, True)
            s_wait(0)
            s_wait(1)
            plsc.subcore_barrier()

            @pl.when(c == 0)
            def _():
                _writeback(acc, outlo, s, b * _N)

            @pl.when(c == 1)
            def _():
                _writeback(acc, outhi, s, b * _N)
            plsc.subcore_barrier()
            return carry
        lax.fori_loop(0, _B, batch, 0)

    return pl.kernel(
        body,
        out_type=(jax.ShapeDtypeStruct((_R, 128), jnp.float32),
                  jax.ShapeDtypeStruct((_R, 128), jnp.float32)),
        mesh=_sc_mesh(),
        scratch_types=(
            pltpu.VMEM((_K,), jnp.int32),
            pltpu.VMEM((_K,), jnp.int32),
            pltpu.VMEM((1, _K), jnp.int32),
            pltpu.VMEM((1, _K), jnp.int32),
            pltpu.VMEM((1, _K), jnp.int32),
            pltpu.VMEM((1, _K), jnp.int32),
            pltpu.VMEM((_K, 16), jnp.float32),
            pltpu.VMEM((_K, 16), jnp.float32),
            pltpu.VMEM((_K,), jnp.int32),
            pltpu.VMEM((_K,), jnp.int32),
            pltpu.VMEM((_K, 128), jnp.float32),
            pltpu.VMEM((_K, 128), jnp.float32),
            pltpu.VMEM_SHARED((_N, 128), jnp.float32),
            pltpu.SemaphoreType.DMA,
            pltpu.SemaphoreType.DMA,
            pltpu.SemaphoreType.DMA,
            pltpu.SemaphoreType.DMA,
            pltpu.SemaphoreType.DMA,
            pltpu.SemaphoreType.DMA,
        ),
        name="sc_spmm_fs%d" % int(feature_split),
    )


def _make_deg():
    nsl = _NC * _NS
    nch = _E // nsl // _K
    zrows = 80

    def body(dst_hbm, ewb_hbm, outlo, outhi, dstc, wbuf, wfull, acc, wsem):
        c = lax.axis_index("c")
        s = lax.axis_index("s")
        w = c * _NS + s
        zv = jnp.zeros((16,), jnp.float32)

        def zinit(r, carry0):
            for f in range(8):
                wfull[r, pl.ds(f * 16, 16)] = zv
            return carry0
        lax.fori_loop(0, zrows, zinit, 0)
        _zero_acc(wfull, acc, s, zrows)
        plsc.subcore_barrier()

        def chunk(i, carry2):
            cpd = pltpu.async_copy(dst_hbm.at[w, i], dstc.at[0], wsem)
            cpw = pltpu.async_copy(ewb_hbm.at[w, i], wbuf, wsem)
            cpd.wait()
            cpw.wait()

            def rowloop(j, carry3):
                wv = wbuf[j, pl.ds(0, 16)]
                for f in range(8):
                    wfull[j, pl.ds(f * 16, 16)] = wv
                return carry3
            lax.fori_loop(0, _K, rowloop, 0)
            pltpu.sync_copy(wfull, acc.at[dstc.at[0]], add=True)
            return carry2
        lax.fori_loop(0, nch, chunk, 0)
        plsc.subcore_barrier()

        @pl.when(c == 0)
        def _():
            _writeback(acc, outlo, s, 0)

        @pl.when(c == 1)
        def _():
            _writeback(acc, outhi, s, 0)

    return pl.kernel(
        body,
        out_type=(jax.ShapeDtypeStruct((_N, 128), jnp.float32),
                  jax.ShapeDtypeStruct((_N, 128), jnp.float32)),
        mesh=_sc_mesh(),
        scratch_types=(
            pltpu.VMEM((1, _K), jnp.int32),
            pltpu.VMEM((_K, 16), jnp.float32),
            pltpu.VMEM((_K, 128), jnp.float32),
            pltpu.VMEM_SHARED((_N, 128), jnp.float32),
            pltpu.SemaphoreType.DMA,
        ),
        name="sc_deg",
    )


_deg_kernel = _make_deg()
_spmm_l0 = _make_spmm(False)
_spmm_l1 = _make_spmm(True)


_BLK = 2000


def _combine0_body(alo, ahi, hb, sc, wl, wr, bb, out):
    agg = sc[...] * (alo[...] + ahi[...])
    t = jnp.dot(agg, wl[...], preferred_element_type=jnp.float32)
    t = t + jnp.dot(hb[...], wr[...], preferred_element_type=jnp.float32)
    t = jnp.maximum(t + bb[...], 0.0)
    out[...] = t.reshape(2 * _BLK, 128)


def _combine0(a0lo, a0hi, h2, scale, W_lin, W_root, b):
    g = _R // _BLK
    return pl.pallas_call(
        _combine0_body,
        grid=(g,),
        in_specs=[
            pl.BlockSpec((_BLK, 128), lambda i: (i, 0)),
            pl.BlockSpec((_BLK, 128), lambda i: (i, 0)),
            pl.BlockSpec((_BLK, 128), lambda i: (i, 0)),
            pl.BlockSpec((_BLK, 1), lambda i: (i, 0)),
            pl.BlockSpec((128, 256), lambda i: (0, 0)),
            pl.BlockSpec((128, 256), lambda i: (0, 0)),
            pl.BlockSpec((1, 256), lambda i: (0, 0)),
        ],
        out_specs=pl.BlockSpec((2 * _BLK, 128), lambda i: (i, 0)),
        out_shape=jax.ShapeDtypeStruct((2 * _R, 128), jnp.float32),
    )(a0lo, a0hi, h2, scale, W_lin, W_root, b)


def _mlp_body(alo, ahi, x1b, sc, wl, wr, bb, wm, bm_, wrd, brd, out):
    a = jnp.dot(sc[...] * alo[...], wl[0:128, :],
                preferred_element_type=jnp.float32)
    a = a + jnp.dot(sc[...] * ahi[...], wl[128:256, :],
                    preferred_element_type=jnp.float32)
    x1 = x1b[...].reshape(_BLK, 256)
    t = a + jnp.dot(x1, wr[...], preferred_element_type=jnp.float32)
    t = jnp.maximum(t + bb[...], 0.0)
    t2 = jnp.dot(t, wm[...], preferred_element_type=jnp.float32)
    t2 = jnp.maximum(t2 + bm_[...], 0.0)
    o = jnp.dot(t2, wrd[...], preferred_element_type=jnp.float32)
    out[...] = o + brd[...]


def _mlp(a1lo, a1hi, x1i, scale, W_lin, W_root, b1, Wm, bm, Wr, br):
    g = _R // _BLK
    return pl.pallas_call(
        _mlp_body,
        grid=(g,),
        in_specs=[
            pl.BlockSpec((_BLK, 128), lambda i: (i, 0)),
            pl.BlockSpec((_BLK, 128), lambda i: (i, 0)),
            pl.BlockSpec((2 * _BLK, 128), lambda i: (i, 0)),
            pl.BlockSpec((_BLK, 1), lambda i: (i, 0)),
            pl.BlockSpec((256, 256), lambda i: (0, 0)),
            pl.BlockSpec((256, 256), lambda i: (0, 0)),
            pl.BlockSpec((1, 256), lambda i: (0, 0)),
            pl.BlockSpec((256, 256), lambda i: (0, 0)),
            pl.BlockSpec((1, 256), lambda i: (0, 0)),
            pl.BlockSpec((256, _HOR), lambda i: (0, 0)),
            pl.BlockSpec((1, _HOR), lambda i: (0, 0)),
        ],
        out_specs=pl.BlockSpec((_BLK, _HOR), lambda i: (i, 0)),
        out_shape=jax.ShapeDtypeStruct((_R, _HOR), jnp.float32),
    )(a1lo, a1hi, x1i, scale, W_lin, W_root, b1, Wm, bm, Wr, br)


def kernel(h, edge_index, edge_weight, W0_lin, W0_root, b0,
           W1_lin, W1_root, b1, Wm, bm, Wr, br):
    src = edge_index[0]
    dst = edge_index[1]
    ewb = jnp.broadcast_to(edge_weight[:, None], (_E, 16))
    n16 = _E // _NS // _K
    src16 = src.reshape(_NS, n16, _K)
    dst16 = dst.reshape(_NS, n16, _K)
    ewb16 = ewb.reshape(_NS, n16, _K, 16)
    n32 = _E // (_NC * _NS) // _K
    src32 = src.reshape(_NC * _NS, n32, _K)
    dst32 = dst.reshape(_NC * _NS, n32, _K)
    ewb32 = ewb.reshape(_NC * _NS, n32, _K, 16)

    deglo, deghi = _deg_kernel(dst32, ewb32)
    deg = deglo[:, 0] + deghi[:, 0]
    inv = jnp.where(deg > 0, 1.0 / deg, 0.0)
    scale = jnp.broadcast_to(inv[None, :], (_B, _N)).reshape(_R, 1)

    h2 = h.reshape(_R, 128)
    a0lo, a0hi = _spmm_l0(h2, src32, dst32, ewb32)
    x1_il = _combine0(a0lo, a0hi, h2, scale, W0_lin, W0_root,
                      b0.reshape(1, 256))
    a1lo, a1hi = _spmm_l1(x1_il, src16, dst16, ewb16)
    o = _mlp(a1lo, a1hi, x1_il, scale, W1_lin, W1_root, b1.reshape(1, 256),
             Wm, bm.reshape(1, 256), Wr, br.reshape(1, _HOR))
    return o.reshape(_B, _N, _HOR, 1).transpose(0, 2, 1, 3)


# speedup vs baseline: 14.7916x; 1.0001x over previous
import jax
import jax.numpy as jnp
from jax import lax
from jax.experimental import pallas as pl
from jax.experimental.pallas import tpu as pltpu
from jax.experimental.pallas import tpu_sc as plsc

_B = 8
_N = 10000
_E = 320000
_R = _B * _N
_NC = 2
_NS = 16
_K = 80
_HOR = 12
_W0 = 640
_W1 = 400


def _sc_mesh():
    return plsc.VectorSubcoreMesh(
        core_axis_name="c", subcore_axis_name="s",
        num_cores=_NC, num_subcores=_NS)


def _zero_acc(zbuf, acc, s, zrows):
    @pl.when(s < _NS - 1)
    def _():
        for z in range(_W0 // zrows):
            pltpu.sync_copy(zbuf, acc.at[pl.ds(s * _W0 + z * zrows, zrows)])

    @pl.when(s == _NS - 1)
    def _():
        for z in range(_W1 // zrows):
            pltpu.sync_copy(zbuf, acc.at[pl.ds(s * _W0 + z * zrows, zrows)])


def _writeback(acc, out, s, row0):
    @pl.when(s < _NS - 1)
    def _():
        pltpu.sync_copy(acc.at[pl.ds(s * _W0, _W0)],
                        out.at[pl.ds(row0 + s * _W0, _W0)])

    @pl.when(s == _NS - 1)
    def _():
        pltpu.sync_copy(acc.at[pl.ds(s * _W0, _W1)],
                        out.at[pl.ds(row0 + s * _W0, _W1)])


def _make_spmm(feature_split):
    nsl = _NS if feature_split else _NC * _NS
    nch = _E // nsl // _K
    mul = 2 if feature_split else 1
    zrows = 80
    kmax = (nch - 1) // 2
    tail = (nch % 2 == 0)

    def body(xi, src_hbm, dst_hbm, ewb_hbm, outlo, outhi,
             srcb0, srcb1, dstc0, dstc1, sidx0, sidx1, wbuf0, wbuf1,
             idxb0, idxb1, rows0, rows1, acc,
             esem0, esem1, gsem0, gsem1, ssem0, ssem1):
        c = lax.axis_index("c")
        s = lax.axis_index("s")
        w = s if feature_split else c * _NS + s
        zv = jnp.zeros((16,), jnp.float32)
        srcb = (srcb0, srcb1)
        dstc = (dstc0, dstc1)
        sidx = (sidx0, sidx1)
        wbuf = (wbuf0, wbuf1)
        idxb = (idxb0, idxb1)
        rows = (rows0, rows1)
        esem = (esem0, esem1)
        gsem = (gsem0, gsem1)
        ssem = (ssem0, ssem1)

        def e_issue_sd(j, p):
            pltpu.async_copy(src_hbm.at[w, j], srcb[p], esem[p])
            pltpu.async_copy(dst_hbm.at[w, j], dstc[p].at[0], esem[p])

        def e_issue_w(j, p):
            pltpu.async_copy(ewb_hbm.at[w, j], wbuf[p], esem[p])

        def e_wait(p):
            pltpu.make_async_copy(src_hbm.at[w, 0], srcb[p], esem[p]).wait()
            pltpu.make_async_copy(dst_hbm.at[w, 0], dstc[p].at[0],
                                  esem[p]).wait()
            pltpu.make_async_copy(ewb_hbm.at[w, 0], wbuf[p], esem[p]).wait()

        def g_issue(p):
            pltpu.async_copy(xi.at[idxb[p]], rows[p], gsem[p])

        def g_wait(p):
            pltpu.make_async_copy(xi.at[idxb[p]], rows[p], gsem[p]).wait()

        def s_issue(p):
            pltpu.async_copy(rows[p], acc.at[sidx[p].at[0]], ssem[p],
                             add=True)

        def s_wait(p):
            pltpu.make_async_copy(rows[p], acc.at[sidx[p].at[0]],
                                  ssem[p]).wait()

        def scale(p):
            def rowloop(g, carry3):
                base = g * 8
                for r in range(8):
                    j = base + r
                    wv = wbuf[p][j, pl.ds(0, 16)]
                    for f in range(8):
                        sl = pl.ds(f * 16, 16)
                        rows[p][j, sl] = rows[p][j, sl] * wv
                return carry3
            lax.fori_loop(0, _K // 8, rowloop, 0)

        def batch(b, carry):
            def zinit(r, carry0):
                for f in range(8):
                    rows[0][r, pl.ds(f * 16, 16)] = zv
                return carry0
            lax.fori_loop(0, zrows, zinit, 0)
            _zero_acc(rows[0], acc, s, zrows)
            plsc.subcore_barrier()
            if feature_split:
                off = b * (2 * _N) + c
            else:
                off = b * _N

            def build_idx(p):
                for t in range(_K // 16):
                    sv = srcb[p][pl.ds(t * 16, 16)]
                    idxb[p][pl.ds(t * 16, 16)] = sv * mul + off

            def copy_sidx(p):
                for t in range(_K // 16):
                    sidx[p][0, pl.ds(t * 16, 16)] = \
                        dstc[p][0, pl.ds(t * 16, 16)]

            def pref(jn, p, first):
                e_wait(p)
                build_idx(p)
                if not first:
                    s_wait(p)
                copy_sidx(p)
                g_issue(p)

                @pl.when(jn + 2 < nch)
                def _():
                    e_issue_sd(jn + 2, p)

            def proc(j, p, last):
                g_wait(p)
                scale(p)
                if not last:
                    @pl.when(j + 2 < nch)
                    def _():
                        e_issue_w(j + 2, p)
                s_issue(p)

            e_issue_sd(0, 0)
            e_issue_w(0, 0)
            e_issue_sd(1, 1)
            e_issue_w(1, 1)
            pref(0, 0, True)
            pref(1, 1, True)
            proc(0, 0, False)

            def pair(k, carry2):
                j0 = 2 * k - 1

                @pl.when(j0 + 1 < nch)
                def _():
                    pref(j0 + 1, 0, False)
                proc(j0, 1, False)

                @pl.when(j0 + 2 < nch)
                def _():
                    pref(j0 + 2, 1, False)
                proc(j0 + 1, 0, False)
                return carry2
            lax.fori_loop(1, kmax + 1, pair, 0)
            if tail:
                proc(nch - 1, 1, True)
            s_wait(0)
            s_wait(1)
            plsc.subcore_barrier()

            @pl.when(c == 0)
            def _():
                _writeback(acc, outlo, s, b * _N)

            @pl.when(c == 1)
            def _():
                _writeback(acc, outhi, s, b * _N)
            plsc.subcore_barrier()
            return carry
        lax.fori_loop(0, _B, batch, 0)

    return pl.kernel(
        body,
        out_type=(jax.ShapeDtypeStruct((_R, 128), jnp.float32),
                  jax.ShapeDtypeStruct((_R, 128), jnp.float32)),
        mesh=_sc_mesh(),
        scratch_types=(
            pltpu.VMEM((_K,), jnp.int32),
            pltpu.VMEM((_K,), jnp.int32),
            pltpu.VMEM((1, _K), jnp.int32),
            pltpu.VMEM((1, _K), jnp.int32),
            pltpu.VMEM((1, _K), jnp.int32),
            pltpu.VMEM((1, _K), jnp.int32),
            pltpu.VMEM((_K, 16), jnp.float32),
            pltpu.VMEM((_K, 16), jnp.float32),
            pltpu.VMEM((_K,), jnp.int32),
            pltpu.VMEM((_K,), jnp.int32),
            pltpu.VMEM((_K, 128), jnp.float32),
            pltpu.VMEM((_K, 128), jnp.float32),
            pltpu.VMEM_SHARED((_N, 128), jnp.float32),
            pltpu.SemaphoreType.DMA,
            pltpu.SemaphoreType.DMA,
            pltpu.SemaphoreType.DMA,
            pltpu.SemaphoreType.DMA,
            pltpu.SemaphoreType.DMA,
            pltpu.SemaphoreType.DMA,
        ),
        name="sc_spmm_fs%d" % int(feature_split),
    )


def _make_deg():
    nsl = _NC * _NS
    nch = _E // nsl // _K
    zrows = 80

    def body(dst_hbm, ewb_hbm, outlo, outhi, dstc, wbuf, wfull, acc, wsem):
        c = lax.axis_index("c")
        s = lax.axis_index("s")
        w = c * _NS + s
        zv = jnp.zeros((16,), jnp.float32)

        def zinit(r, carry0):
            for f in range(8):
                wfull[r, pl.ds(f * 16, 16)] = zv
            return carry0
        lax.fori_loop(0, zrows, zinit, 0)
        _zero_acc(wfull, acc, s, zrows)
        plsc.subcore_barrier()

        def chunk(i, carry2):
            cpd = pltpu.async_copy(dst_hbm.at[w, i], dstc.at[0], wsem)
            cpw = pltpu.async_copy(ewb_hbm.at[w, i], wbuf, wsem)
            cpd.wait()
            cpw.wait()

            def rowloop(j, carry3):
                wv = wbuf[j, pl.ds(0, 16)]
                for f in range(8):
                    wfull[j, pl.ds(f * 16, 16)] = wv
                return carry3
            lax.fori_loop(0, _K, rowloop, 0)
            pltpu.sync_copy(wfull, acc.at[dstc.at[0]], add=True)
            return carry2
        lax.fori_loop(0, nch, chunk, 0)
        plsc.subcore_barrier()

        @pl.when(c == 0)
        def _():
            _writeback(acc, outlo, s, 0)

        @pl.when(c == 1)
        def _():
            _writeback(acc, outhi, s, 0)

    return pl.kernel(
        body,
        out_type=(jax.ShapeDtypeStruct((_N, 128), jnp.float32),
                  jax.ShapeDtypeStruct((_N, 128), jnp.float32)),
        mesh=_sc_mesh(),
        scratch_types=(
            pltpu.VMEM((1, _K), jnp.int32),
            pltpu.VMEM((_K, 16), jnp.float32),
            pltpu.VMEM((_K, 128), jnp.float32),
            pltpu.VMEM_SHARED((_N, 128), jnp.float32),
            pltpu.SemaphoreType.DMA,
        ),
        name="sc_deg",
    )


_deg_kernel = _make_deg()
_spmm_l0 = _make_spmm(False)
_spmm_l1 = _make_spmm(True)


_BLK = 2000


def _combine0_body(alo, ahi, hb, sc, wl, wr, bb, out):
    agg = sc[...] * (alo[...] + ahi[...])
    t = jnp.dot(agg, wl[...], preferred_element_type=jnp.float32)
    t = t + jnp.dot(hb[...], wr[...], preferred_element_type=jnp.float32)
    t = jnp.maximum(t + bb[...], 0.0)
    out[...] = t.reshape(2 * _BLK, 128)


def _combine0(a0lo, a0hi, h2, scale, W_lin, W_root, b):
    g = _R // _BLK
    return pl.pallas_call(
        _combine0_body,
        grid=(g,),
        in_specs=[
            pl.BlockSpec((_BLK, 128), lambda i: (i, 0)),
            pl.BlockSpec((_BLK, 128), lambda i: (i, 0)),
            pl.BlockSpec((_BLK, 128), lambda i: (i, 0)),
            pl.BlockSpec((_BLK, 1), lambda i: (i, 0)),
            pl.BlockSpec((128, 256), lambda i: (0, 0)),
            pl.BlockSpec((128, 256), lambda i: (0, 0)),
            pl.BlockSpec((1, 256), lambda i: (0, 0)),
        ],
        out_specs=pl.BlockSpec((2 * _BLK, 128), lambda i: (i, 0)),
        out_shape=jax.ShapeDtypeStruct((2 * _R, 128), jnp.float32),
    )(a0lo, a0hi, h2, scale, W_lin, W_root, b)


def _mlp_body(alo, ahi, x1b, sc, wl, wr, bb, wm, bm_, wrd, brd, out):
    a = jnp.dot(sc[...] * alo[...], wl[0:128, :],
                preferred_element_type=jnp.float32)
    a = a + jnp.dot(sc[...] * ahi[...], wl[128:256, :],
                    preferred_element_type=jnp.float32)
    x1 = x1b[...].reshape(_BLK, 256)
    t = a + jnp.dot(x1, wr[...], preferred_element_type=jnp.float32)
    t = jnp.maximum(t + bb[...], 0.0)
    t2 = jnp.dot(t, wm[...], preferred_element_type=jnp.float32)
    t2 = jnp.maximum(t2 + bm_[...], 0.0)
    o = jnp.dot(t2, wrd[...], preferred_element_type=jnp.float32)
    out[...] = o + brd[...]


def _mlp(a1lo, a1hi, x1i, scale, W_lin, W_root, b1, Wm, bm, Wr, br):
    g = _R // _BLK
    return pl.pallas_call(
        _mlp_body,
        grid=(g,),
        in_specs=[
            pl.BlockSpec((_BLK, 128), lambda i: (i, 0)),
            pl.BlockSpec((_BLK, 128), lambda i: (i, 0)),
            pl.BlockSpec((2 * _BLK, 128), lambda i: (i, 0)),
            pl.BlockSpec((_BLK, 1), lambda i: (i, 0)),
            pl.BlockSpec((256, 256), lambda i: (0, 0)),
            pl.BlockSpec((256, 256), lambda i: (0, 0)),
            pl.BlockSpec((1, 256), lambda i: (0, 0)),
            pl.BlockSpec((256, 256), lambda i: (0, 0)),
            pl.BlockSpec((1, 256), lambda i: (0, 0)),
            pl.BlockSpec((256, _HOR), lambda i: (0, 0)),
            pl.BlockSpec((1, _HOR), lambda i: (0, 0)),
        ],
        out_specs=pl.BlockSpec((_BLK, _HOR), lambda i: (i, 0)),
        out_shape=jax.ShapeDtypeStruct((_R, _HOR), jnp.float32),
    )(a1lo, a1hi, x1i, scale, W_lin, W_root, b1, Wm, bm, Wr, br)


def kernel(h, edge_index, edge_weight, W0_lin, W0_root, b0,
           W1_lin, W1_root, b1, Wm, bm, Wr, br):
    src = edge_index[0]
    dst = edge_index[1]
    ewb = jnp.broadcast_to(edge_weight[:, None], (_E, 16))
    n16 = _E // _NS // _K
    src16 = src.reshape(_NS, n16, _K)
    dst16 = dst.reshape(_NS, n16, _K)
    ewb16 = ewb.reshape(_NS, n16, _K, 16)
    n32 = _E // (_NC * _NS) // _K
    src32 = src.reshape(_NC * _NS, n32, _K)
    dst32 = dst.reshape(_NC * _NS, n32, _K)
    ewb32 = ewb.reshape(_NC * _NS, n32, _K, 16)

    deglo, deghi = _deg_kernel(dst32, ewb32)
    deg = deglo[:, 0] + deghi[:, 0]
    inv = jnp.where(deg > 0, 1.0 / deg, 0.0)
    scale = jnp.broadcast_to(inv[None, :], (_B, _N)).reshape(_R, 1)

    h2 = h.reshape(_R, 128)
    a0lo, a0hi = _spmm_l0(h2, src32, dst32, ewb32)
    x1_il = _combine0(a0lo, a0hi, h2, scale, W0_lin, W0_root,
                      b0.reshape(1, 256))
    a1lo, a1hi = _spmm_l1(x1_il, src16, dst16, ewb16)
    o = _mlp(a1lo, a1hi, x1_il, scale, W1_lin, W1_root, b1.reshape(1, 256),
             Wm, bm.reshape(1, 256), Wr, br.reshape(1, _HOR))
    return o.reshape(_B, _N, _HOR, 1).transpose(0, 2, 1, 3)


# speedup vs baseline: 18.2420x; 1.2333x over previous
import jax
import jax.numpy as jnp
from jax import lax
from jax.experimental import pallas as pl
from jax.experimental.pallas import tpu as pltpu
from jax.experimental.pallas import tpu_sc as plsc

_B = 8
_N = 10000
_E = 320000
_R = _B * _N
_NC = 2
_NS = 16
_K = 80
_HOR = 12
_W0 = 640
_W1 = 400


def _sc_mesh():
    return plsc.VectorSubcoreMesh(
        core_axis_name="c", subcore_axis_name="s",
        num_cores=_NC, num_subcores=_NS)


def _zero_acc(zbuf, acc, s, zrows):
    @pl.when(s < _NS - 1)
    def _():
        for z in range(_W0 // zrows):
            pltpu.sync_copy(zbuf, acc.at[pl.ds(s * _W0 + z * zrows, zrows)])

    @pl.when(s == _NS - 1)
    def _():
        for z in range(_W1 // zrows):
            pltpu.sync_copy(zbuf, acc.at[pl.ds(s * _W0 + z * zrows, zrows)])


def _writeback(acc, out, s, row0):
    @pl.when(s < _NS - 1)
    def _():
        pltpu.sync_copy(acc.at[pl.ds(s * _W0, _W0)],
                        out.at[pl.ds(row0 + s * _W0, _W0)])

    @pl.when(s == _NS - 1)
    def _():
        pltpu.sync_copy(acc.at[pl.ds(s * _W0, _W1)],
                        out.at[pl.ds(row0 + s * _W0, _W1)])


def _make_spmm(feature_split):
    nsl = _NS if feature_split else _NC * _NS
    nch = _E // nsl // _K
    mul = 2 if feature_split else 1
    zrows = 80
    kmax = (nch - 1) // 2
    tail = (nch % 2 == 0)

    def body(xi, src_hbm, dst_hbm, ewb_hbm, outlo, outhi,
             srcb0, srcb1, sidx0, sidx1, wbuf0, wbuf1,
             idxb0, idxb1, rows0, rows1, acc,
             esem0, esem1, wsem0, wsem1, gsem0, gsem1, ssem0, ssem1):
        c = lax.axis_index("c")
        s = lax.axis_index("s")
        w = s if feature_split else c * _NS + s
        zv = jnp.zeros((16,), jnp.float32)
        srcb = (srcb0, srcb1)
        sidx = (sidx0, sidx1)
        wbuf = (wbuf0, wbuf1)
        idxb = (idxb0, idxb1)
        rows = (rows0, rows1)
        esem = (esem0, esem1)
        wsem = (wsem0, wsem1)
        gsem = (gsem0, gsem1)
        ssem = (ssem0, ssem1)

        def sd_issue(j, p):
            pltpu.async_copy(src_hbm.at[w, j], srcb[p], esem[p])
            pltpu.async_copy(dst_hbm.at[w, j], sidx[p].at[1], esem[p])

        def sd_wait(p):
            pltpu.make_async_copy(src_hbm.at[w, 0], srcb[p], esem[p]).wait()
            pltpu.make_async_copy(dst_hbm.at[w, 0], sidx[p].at[1],
                                  esem[p]).wait()

        def w_issue(j, p):
            pltpu.async_copy(ewb_hbm.at[w, j], wbuf[p], wsem[p])

        def w_wait(p):
            pltpu.make_async_copy(ewb_hbm.at[w, 0], wbuf[p], wsem[p]).wait()

        def g_issue(p):
            pltpu.async_copy(xi.at[idxb[p]], rows[p], gsem[p])

        def g_wait(p):
            pltpu.make_async_copy(xi.at[idxb[p]], rows[p], gsem[p]).wait()

        def s_issue(p):
            pltpu.async_copy(rows[p], acc.at[sidx[p].at[0]], ssem[p],
                             add=True)

        def s_wait(p):
            pltpu.make_async_copy(rows[p], acc.at[sidx[p].at[0]],
                                  ssem[p]).wait()

        def scale(p):
            def rowloop(g, carry3):
                base = g * 8
                for r in range(8):
                    j = base + r
                    wv = wbuf[p][j, pl.ds(0, 16)]
                    for f in range(8):
                        sl = pl.ds(f * 16, 16)
                        rows[p][j, sl] = rows[p][j, sl] * wv
                return carry3
            lax.fori_loop(0, _K // 8, rowloop, 0)

        def batch(b, carry):
            def zinit(r, carry0):
                for f in range(8):
                    rows[0][r, pl.ds(f * 16, 16)] = zv
                return carry0
            lax.fori_loop(0, zrows, zinit, 0)
            _zero_acc(rows[0], acc, s, zrows)
            plsc.subcore_barrier()
            if feature_split:
                off = b * (2 * _N) + c
            else:
                off = b * _N

            def build_idx(p):
                for t in range(_K // 16):
                    sv = srcb[p][pl.ds(t * 16, 16)]
                    idxb[p][pl.ds(t * 16, 16)] = sv * mul + off

            def copy_sidx(p):
                for t in range(_K // 16):
                    sidx[p][0, pl.ds(t * 16, 16)] = \
                        sidx[p][1, pl.ds(t * 16, 16)]

            def pref(jn, p, first):
                sd_wait(p)
                build_idx(p)
                if not first:
                    s_wait(p)
                copy_sidx(p)
                g_issue(p)

                @pl.when(jn + 2 < nch)
                def _():
                    sd_issue(jn + 2, p)

            def proc(j, p, last):
                g_wait(p)
                w_wait(p)
                scale(p)
                if not last:
                    @pl.when(j + 2 < nch)
                    def _():
                        w_issue(j + 2, p)
                s_issue(p)

            sd_issue(0, 0)
            w_issue(0, 0)
            sd_issue(1, 1)
            w_issue(1, 1)
            pref(0, 0, True)
            pref(1, 1, True)
            proc(0, 0, False)

            def pair(k, carry2):
                j0 = 2 * k - 1

                @pl.when(j0 + 1 < nch)
                def _():
                    pref(j0 + 1, 0, False)
                proc(j0, 1, False)

                @pl.when(j0 + 2 < nch)
                def _():
                    pref(j0 + 2, 1, False)
                proc(j0 + 1, 0, False)
                return carry2
            lax.fori_loop(1, kmax + 1, pair, 0)
            if tail:
                proc(nch - 1, 1, True)
            s_wait(0)
            s_wait(1)
            plsc.subcore_barrier()

            @pl.when(c == 0)
            def _():
                _writeback(acc, outlo, s, b * _N)

            @pl.when(c == 1)
            def _():
                _writeback(acc, outhi, s, b * _N)
            plsc.subcore_barrier()
            return carry
        lax.fori_loop(0, _B, batch, 0)

    return pl.kernel(
        body,
        out_type=(jax.ShapeDtypeStruct((_R, 128), jnp.float32),
                  jax.ShapeDtypeStruct((_R, 128), jnp.float32)),
        mesh=_sc_mesh(),
        scratch_types=(
            pltpu.VMEM((_K,), jnp.int32),
            pltpu.VMEM((_K,), jnp.int32),
            pltpu.VMEM((2, _K), jnp.int32),
            pltpu.VMEM((2, _K), jnp.int32),
            pltpu.VMEM((_K, 16), jnp.float32),
            pltpu.VMEM((_K, 16), jnp.float32),
            pltpu.VMEM((_K,), jnp.int32),
            pltpu.VMEM((_K,), jnp.int32),
            pltpu.VMEM((_K, 128), jnp.float32),
            pltpu.VMEM((_K, 128), jnp.float32),
            pltpu.VMEM_SHARED((_N, 128), jnp.float32),
            pltpu.SemaphoreType.DMA,
            pltpu.SemaphoreType.DMA,
            pltpu.SemaphoreType.DMA,
            pltpu.SemaphoreType.DMA,
            pltpu.SemaphoreType.DMA,
            pltpu.SemaphoreType.DMA,
            pltpu.SemaphoreType.DMA,
            pltpu.SemaphoreType.DMA,
        ),
        name="sc_spmm_fs%d" % int(feature_split),
    )


def _make_deg():
    nsl = _NC * _NS
    nch = _E // nsl // _K
    zrows = 80

    def body(dst_hbm, ewb_hbm, outlo, outhi, dstc, wbuf, wfull, acc, wsem):
        c = lax.axis_index("c")
        s = lax.axis_index("s")
        w = c * _NS + s
        zv = jnp.zeros((16,), jnp.float32)

        def zinit(r, carry0):
            for f in range(8):
                wfull[r, pl.ds(f * 16, 16)] = zv
            return carry0
        lax.fori_loop(0, zrows, zinit, 0)
        _zero_acc(wfull, acc, s, zrows)
        plsc.subcore_barrier()

        def chunk(i, carry2):
            cpd = pltpu.async_copy(dst_hbm.at[w, i], dstc.at[0], wsem)
            cpw = pltpu.async_copy(ewb_hbm.at[w, i], wbuf, wsem)
            cpd.wait()
            cpw.wait()

            def rowloop(j, carry3):
                wv = wbuf[j, pl.ds(0, 16)]
                for f in range(8):
                    wfull[j, pl.ds(f * 16, 16)] = wv
                return carry3
            lax.fori_loop(0, _K, rowloop, 0)
            pltpu.sync_copy(wfull, acc.at[dstc.at[0]], add=True)
            return carry2
        lax.fori_loop(0, nch, chunk, 0)
        plsc.subcore_barrier()

        @pl.when(c == 0)
        def _():
            _writeback(acc, outlo, s, 0)

        @pl.when(c == 1)
        def _():
            _writeback(acc, outhi, s, 0)

    return pl.kernel(
        body,
        out_type=(jax.ShapeDtypeStruct((_N, 128), jnp.float32),
                  jax.ShapeDtypeStruct((_N, 128), jnp.float32)),
        mesh=_sc_mesh(),
        scratch_types=(
            pltpu.VMEM((1, _K), jnp.int32),
            pltpu.VMEM((_K, 16), jnp.float32),
            pltpu.VMEM((_K, 128), jnp.float32),
            pltpu.VMEM_SHARED((_N, 128), jnp.float32),
            pltpu.SemaphoreType.DMA,
        ),
        name="sc_deg",
    )


_deg_kernel = _make_deg()
_spmm_l0 = _make_spmm(False)
_spmm_l1 = _make_spmm(True)


_BLK = 2000


def _combine0_body(alo, ahi, hb, sc, wl, wr, bb, out):
    agg = sc[...] * (alo[...] + ahi[...])
    t = jnp.dot(agg, wl[...], preferred_element_type=jnp.float32)
    t = t + jnp.dot(hb[...], wr[...], preferred_element_type=jnp.float32)
    t = jnp.maximum(t + bb[...], 0.0)
    out[...] = t.reshape(2 * _BLK, 128)


def _combine0(a0lo, a0hi, h2, scale, W_lin, W_root, b):
    g = _R // _BLK
    return pl.pallas_call(
        _combine0_body,
        grid=(g,),
        in_specs=[
            pl.BlockSpec((_BLK, 128), lambda i: (i, 0)),
            pl.BlockSpec((_BLK, 128), lambda i: (i, 0)),
            pl.BlockSpec((_BLK, 128), lambda i: (i, 0)),
            pl.BlockSpec((_BLK, 1), lambda i: (i, 0)),
            pl.BlockSpec((128, 256), lambda i: (0, 0)),
            pl.BlockSpec((128, 256), lambda i: (0, 0)),
            pl.BlockSpec((1, 256), lambda i: (0, 0)),
        ],
        out_specs=pl.BlockSpec((2 * _BLK, 128), lambda i: (i, 0)),
        out_shape=jax.ShapeDtypeStruct((2 * _R, 128), jnp.float32),
    )(a0lo, a0hi, h2, scale, W_lin, W_root, b)


def _mlp_body(alo, ahi, x1b, sc, wl, wr, bb, wm, bm_, wrd, brd, out):
    a = jnp.dot(sc[...] * alo[...], wl[0:128, :],
                preferred_element_type=jnp.float32)
    a = a + jnp.dot(sc[...] * ahi[...], wl[128:256, :],
                    preferred_element_type=jnp.float32)
    x1 = x1b[...].reshape(_BLK, 256)
    t = a + jnp.dot(x1, wr[...], preferred_element_type=jnp.float32)
    t = jnp.maximum(t + bb[...], 0.0)
    t2 = jnp.dot(t, wm[...], preferred_element_type=jnp.float32)
    t2 = jnp.maximum(t2 + bm_[...], 0.0)
    o = jnp.dot(t2, wrd[...], preferred_element_type=jnp.float32)
    out[...] = o + brd[...]


def _mlp(a1lo, a1hi, x1i, scale, W_lin, W_root, b1, Wm, bm, Wr, br):
    g = _R // _BLK
    return pl.pallas_call(
        _mlp_body,
        grid=(g,),
        in_specs=[
            pl.BlockSpec((_BLK, 128), lambda i: (i, 0)),
            pl.BlockSpec((_BLK, 128), lambda i: (i, 0)),
            pl.BlockSpec((2 * _BLK, 128), lambda i: (i, 0)),
            pl.BlockSpec((_BLK, 1), lambda i: (i, 0)),
            pl.BlockSpec((256, 256), lambda i: (0, 0)),
            pl.BlockSpec((256, 256), lambda i: (0, 0)),
            pl.BlockSpec((1, 256), lambda i: (0, 0)),
            pl.BlockSpec((256, 256), lambda i: (0, 0)),
            pl.BlockSpec((1, 256), lambda i: (0, 0)),
            pl.BlockSpec((256, _HOR), lambda i: (0, 0)),
            pl.BlockSpec((1, _HOR), lambda i: (0, 0)),
        ],
        out_specs=pl.BlockSpec((_BLK, _HOR), lambda i: (i, 0)),
        out_shape=jax.ShapeDtypeStruct((_R, _HOR), jnp.float32),
    )(a1lo, a1hi, x1i, scale, W_lin, W_root, b1, Wm, bm, Wr, br)


def kernel(h, edge_index, edge_weight, W0_lin, W0_root, b0,
           W1_lin, W1_root, b1, Wm, bm, Wr, br):
    src = edge_index[0]
    dst = edge_index[1]
    ewb = jnp.broadcast_to(edge_weight[:, None], (_E, 16))
    n16 = _E // _NS // _K
    src16 = src.reshape(_NS, n16, _K)
    dst16 = dst.reshape(_NS, n16, _K)
    ewb16 = ewb.reshape(_NS, n16, _K, 16)
    n32 = _E // (_NC * _NS) // _K
    src32 = src.reshape(_NC * _NS, n32, _K)
    dst32 = dst.reshape(_NC * _NS, n32, _K)
    ewb32 = ewb.reshape(_NC * _NS, n32, _K, 16)

    deglo, deghi = _deg_kernel(dst32, ewb32)
    deg = deglo[:, 0] + deghi[:, 0]
    inv = jnp.where(deg > 0, 1.0 / deg, 0.0)
    scale = jnp.broadcast_to(inv[None, :], (_B, _N)).reshape(_R, 1)

    h2 = h.reshape(_R, 128)
    a0lo, a0hi = _spmm_l0(h2, src32, dst32, ewb32)
    x1_il = _combine0(a0lo, a0hi, h2, scale, W0_lin, W0_root,
                      b0.reshape(1, 256))
    a1lo, a1hi = _spmm_l1(x1_il, src16, dst16, ewb16)
    o = _mlp(a1lo, a1hi, x1_il, scale, W1_lin, W1_root, b1.reshape(1, 256),
             Wm, bm.reshape(1, 256), Wr, br.reshape(1, _HOR))
    return o.reshape(_B, _N, _HOR, 1).transpose(0, 2, 1, 3)
